# baseline (speedup 1.0000x reference)
import jax
import jax.numpy as jnp
from jax.experimental import pallas as pl
from jax.experimental.pallas import tpu as pltpu
from jax.experimental.pallas import tpu_sc as plsc

N = 8192
D = 1024
NH = 2
H = N // NH
TR = 1024
TE = 512
NBH = H + TE
NTH = NBH // TE
RC = 128
RWH = H // RC
OW = 128
SC_W = 128
SC_C = 4
DC = D // SC_C
SC_GW = 128

_PAR = pltpu.CompilerParams(dimension_semantics=("parallel",))


def _ln_relu(h, g, beta):
    n = h.shape[1]
    ones = jnp.full((n, 1), 1.0 / n, jnp.float32)
    mu = jnp.dot(h, ones, preferred_element_type=jnp.float32)
    hc = h - mu
    var = jnp.dot(hc * hc, ones, preferred_element_type=jnp.float32)
    h = hc * jax.lax.rsqrt(var + 1e-5) * g + beta
    return jnp.maximum(h, 0.0)


def _router_body(x_ref, w1_ref, b1_ref, g_ref, beta_ref, w2_ref, b2_ref,
                 route_ref):
    x = x_ref[...]
    h = jnp.dot(x, w1_ref[...], preferred_element_type=jnp.float32)
    h = _ln_relu(h + b1_ref[...], g_ref[...], beta_ref[...])
    l = jnp.dot(h, w2_ref[...], preferred_element_type=jnp.float32)
    l = l + b2_ref[...]
    route = (l[:, 1:2] > l[:, 0:1]).astype(jnp.int32)
    route_ref[...] = route.reshape(TR // RC, RC)


def _pos_body(r_ref, pos_ref, pos4_ref, te_ref):
    rows = r_ref.shape[0]
    r = r_ref[...]
    isf = (r == 0).astype(jnp.float32)
    ri = jax.lax.broadcasted_iota(jnp.int32, (RC, RC), 0)
    ci = jax.lax.broadcasted_iota(jnp.int32, (RC, RC), 1)
    tri = (ri <= ci).astype(jnp.float32)
    p = jnp.dot(isf, tri, preferred_element_type=jnp.float32)
    s = p[:, RC - 1:RC]
    r2 = jax.lax.broadcasted_iota(jnp.int32, (rows, rows), 0)
    c2 = jax.lax.broadcasted_iota(jnp.int32, (rows, rows), 1)
    lower = (c2 < r2).astype(jnp.float32)
    row_excl = jnp.dot(lower, s, preferred_element_type=jnp.float32)
    fake_incl = (p + row_excl).astype(jnp.int32)
    n_fake = fake_incl[rows - 1:rows, RC - 1:RC]
    f_base = jnp.bitwise_and(n_fake + (TE - 1), -TE)
    gidx = (jax.lax.broadcasted_iota(jnp.int32, (rows, RC), 0) * RC
            + jax.lax.broadcasted_iota(jnp.int32, (rows, RC), 1))
    real_incl = gidx + 1 - fake_incl
    pos = jnp.where(r == 0, fake_incl - 1, f_base + real_incl - 1)
    pos_ref[...] = pos
    chunk = jax.lax.broadcasted_iota(jnp.int32, (SC_C, rows, RC), 0)
    pos4_ref[...] = NBH * chunk + pos[None]
    t = (jax.lax.broadcasted_iota(jnp.int32, (8, 64), 0) * 64
         + jax.lax.broadcasted_iota(jnp.int32, (8, 64), 1))
    te_ref[...] = ((t * TE) >= f_base).astype(jnp.int32)


def _expert_body(te_ref, x0_ref, x1_ref, x2_ref, x3_ref,
                 fw1_ref, fb1_ref, fg_ref, fbeta_ref, fw2_ref, fb2_ref,
                 rw1_ref, rb1_ref, rg_ref, rbeta_ref, rw2_ref, rb2_ref,
                 o_ref):
    e = te_ref[pl.program_id(0)]
    x_refs = (x0_ref, x1_ref, x2_ref, x3_ref)

    def head(w1_ref, b1_ref, g_ref, beta_ref, w2_ref, b2_ref):
        h = jnp.dot(x_refs[0][...], w1_ref[0:DC, :],
                    preferred_element_type=jnp.float32)
        for c in range(1, SC_C):
            h = h + jnp.dot(x_refs[c][...], w1_ref[c * DC:(c + 1) * DC, :],
                            preferred_element_type=jnp.float32)
        h = _ln_relu(h + b1_ref[...], g_ref[...], beta_ref[...])
        l = jnp.dot(h, w2_ref[...], preferred_element_type=jnp.float32)
        l = l + b2_ref[...]
        o_ref[...] = jnp.concatenate(
            [l, jnp.zeros((TE, OW - 2), jnp.float32)], axis=1)

    @pl.when(e == 0)
    def _():
        head(fw1_ref, fb1_ref, fg_ref, fbeta_ref, fw2_ref, fb2_ref)

    @pl.when(e != 0)
    def _():
        head(rw1_ref, rb1_ref, rg_ref, rbeta_ref, rw2_ref, rb2_ref)


def _sc_mesh():
    return plsc.VectorSubcoreMesh(core_axis_name="core",
                                  subcore_axis_name="subcore")


def _sc_scatter_rows(x, pos4, half):
    base = half * (H // SC_W)

    @pl.kernel(out_type=jax.ShapeDtypeStruct((SC_C * NBH, DC), jnp.float32),
               mesh=_sc_mesh())
    def k(x_hbm, i_hbm, o_hbm):
        def body(x_vmem, i_vmem):
            pltpu.sync_copy(x_vmem, o_hbm.at[i_vmem.at[0, 0]])

        pltpu.emit_pipeline(
            body,
            grid=(H // SC_W, SC_C),
            in_specs=[pl.BlockSpec((SC_W, DC), lambda i, c: (base + i, c)),
                      pl.BlockSpec((1, 1, SC_W), lambda i, c: (c, i, 0))],
            out_specs=[],
            core_axis_name=("core", "subcore"),
            dimension_semantics=(pltpu.PARALLEL, pltpu.PARALLEL),
        )(x_hbm, i_hbm)

    return k(x, pos4)


def _sc_gather_rows(data, pos):
    @pl.kernel(out_type=jax.ShapeDtypeStruct((H, OW), jnp.float32),
               mesh=_sc_mesh())
    def k(d_hbm, i_hbm, o_hbm):
        def body(i_vmem, o_vmem):
            pltpu.sync_copy(d_hbm.at[i_vmem.at[0]], o_vmem)

        pltpu.emit_pipeline(
            body,
            grid=(H // SC_GW,),
            in_specs=[pl.BlockSpec((1, SC_GW), lambda i: (i, 0))],
            out_specs=[pl.BlockSpec((SC_GW, OW), lambda i: (i, 0))],
            core_axis_name=("core", "subcore"),
            dimension_semantics=(pltpu.PARALLEL,),
        )(i_hbm, o_hbm)

    return k(data, pos)


def kernel(x,
           first_W1, first_b1, first_g, first_beta, first_W2, first_b2,
           fake_W1, fake_b1, fake_g, fake_beta, fake_W2, fake_b2,
           real_W1, real_b1, real_g, real_beta, real_W2, real_b2):
    f32 = jnp.float32
    full = lambda shape: pl.BlockSpec(shape, lambda i: (0,) * len(shape))

    def router(half):
        base = half * (H // TR)
        return pl.pallas_call(
            _router_body,
            grid=(H // TR,),
            in_specs=[pl.BlockSpec((TR, D), lambda i: (base + i, 0)),
                      full((D, D)), full((1, D)), full((1, D)), full((1, D)),
                      full((D, 2)), full((1, 2))],
            out_specs=pl.BlockSpec((TR // RC, RC), lambda i: (i, 0)),
            out_shape=jax.ShapeDtypeStruct((RWH, RC), jnp.int32),
            compiler_params=_PAR,
        )(x, first_W1, first_b1[None], first_g[None], first_beta[None],
          first_W2, first_b2[None])

    def positions(routes):
        return pl.pallas_call(
            _pos_body,
            out_shape=[jax.ShapeDtypeStruct((RWH, RC), jnp.int32),
                       jax.ShapeDtypeStruct((SC_C, RWH, RC), jnp.int32),
                       jax.ShapeDtypeStruct((8, 64), jnp.int32)],
        )(routes)

    def expert(xs, tile_expert):
        grid_spec = pltpu.PrefetchScalarGridSpec(
            num_scalar_prefetch=1,
            grid=(NTH,),
            in_specs=[
                pl.BlockSpec((TE, DC), lambda i, te: (0 * NTH + i, 0)),
                pl.BlockSpec((TE, DC), lambda i, te: (1 * NTH + i, 0)),
                pl.BlockSpec((TE, DC), lambda i, te: (2 * NTH + i, 0)),
                pl.BlockSpec((TE, DC), lambda i, te: (3 * NTH + i, 0)),
                pl.BlockSpec((D, D), lambda i, te: (0, 0)),
                pl.BlockSpec((1, D), lambda i, te: (0, 0)),
                pl.BlockSpec((1, D), lambda i, te: (0, 0)),
                pl.BlockSpec((1, D), lambda i, te: (0, 0)),
                pl.BlockSpec((D, 2), lambda i, te: (0, 0)),
                pl.BlockSpec((1, 2), lambda i, te: (0, 0)),
                pl.BlockSpec((D, D), lambda i, te: (0, 0)),
                pl.BlockSpec((1, D), lambda i, te: (0, 0)),
                pl.BlockSpec((1, D), lambda i, te: (0, 0)),
                pl.BlockSpec((1, D), lambda i, te: (0, 0)),
                pl.BlockSpec((D, 2), lambda i, te: (0, 0)),
                pl.BlockSpec((1, 2), lambda i, te: (0, 0)),
            ],
            out_specs=pl.BlockSpec((TE, OW), lambda i, te: (i, 0)),
        )
        return pl.pallas_call(
            _expert_body,
            grid_spec=grid_spec,
            out_shape=jax.ShapeDtypeStruct((NBH, OW), f32),
            compiler_params=pltpu.CompilerParams(
                dimension_semantics=("arbitrary",)),
        )(tile_expert, xs, xs, xs, xs,
          fake_W1, fake_b1[None], fake_g[None], fake_beta[None],
          fake_W2, fake_b2[None],
          real_W1, real_b1[None], real_g[None], real_beta[None],
          real_W2, real_b2[None])

    routes_h, pos_h, xs_h = [], [], []
    for h in range(NH):
        routes = router(h)
        pos2, pos4, te2 = positions(routes)
        routes_h.append(routes)
        pos_h.append((pos2, te2))
        xs_h.append(_sc_scatter_rows(x, pos4, h))

    fin_h = []
    for h in range(NH):
        pos2, te2 = pos_h[h]
        out_sorted = expert(xs_h[h], te2.reshape(8 * 64)[:NTH])
        fin_h.append(_sc_gather_rows(out_sorted, pos2)[:, :2])

    route_out = jnp.concatenate([r.reshape(H) for r in routes_h])
    return route_out, jnp.concatenate(fin_h, axis=0)


# speedup vs baseline: 1.1202x; 1.1202x over previous
import jax
import jax.numpy as jnp
from jax.experimental import pallas as pl
from jax.experimental.pallas import tpu as pltpu
from jax.experimental.pallas import tpu_sc as plsc

N = 8192
D = 1024
NH = 1
H = N // NH
TR = 1024
TE = 512
NBH = H + TE
NTH = NBH // TE
RC = 128
RWH = H // RC
OW = 128
SC_W = 128
SC_C = 4
DC = D // SC_C
SC_GW = 128

_PAR = pltpu.CompilerParams(dimension_semantics=("parallel",))


def _ln_relu(h, g, beta):
    mu = jnp.mean(h, axis=-1, keepdims=True)
    var = jnp.mean((h - mu) * (h - mu), axis=-1, keepdims=True)
    h = (h - mu) / jnp.sqrt(var + 1e-5) * g + beta
    return jnp.maximum(h, 0.0)


def _router_body(x_ref, w1_ref, b1_ref, g_ref, beta_ref, w2_ref, b2_ref,
                 route_ref):
    x = x_ref[...]
    h = jnp.dot(x, w1_ref[...], preferred_element_type=jnp.float32)
    h = _ln_relu(h + b1_ref[...], g_ref[...], beta_ref[...])
    l = jnp.dot(h, w2_ref[...], preferred_element_type=jnp.float32)
    l = l + b2_ref[...]
    route = (l[:, 1:2] > l[:, 0:1]).astype(jnp.int32)
    route_ref[...] = route.reshape(TR // RC, RC)


def _pos_body(r_ref, pos_ref, pos4_ref, te_ref):
    rows = r_ref.shape[0]
    r = r_ref[...]
    isf = (r == 0).astype(jnp.float32)
    ri = jax.lax.broadcasted_iota(jnp.int32, (RC, RC), 0)
    ci = jax.lax.broadcasted_iota(jnp.int32, (RC, RC), 1)
    tri = (ri <= ci).astype(jnp.float32)
    p = jnp.dot(isf, tri, preferred_element_type=jnp.float32)
    s = p[:, RC - 1:RC]
    r2 = jax.lax.broadcasted_iota(jnp.int32, (rows, rows), 0)
    c2 = jax.lax.broadcasted_iota(jnp.int32, (rows, rows), 1)
    lower = (c2 < r2).astype(jnp.float32)
    row_excl = jnp.dot(lower, s, preferred_element_type=jnp.float32)
    fake_incl = (p + row_excl).astype(jnp.int32)
    n_fake = fake_incl[rows - 1:rows, RC - 1:RC]
    f_base = jnp.bitwise_and(n_fake + (TE - 1), -TE)
    gidx = (jax.lax.broadcasted_iota(jnp.int32, (rows, RC), 0) * RC
            + jax.lax.broadcasted_iota(jnp.int32, (rows, RC), 1))
    real_incl = gidx + 1 - fake_incl
    pos = jnp.where(r == 0, fake_incl - 1, f_base + real_incl - 1)
    pos_ref[...] = pos
    chunk = jax.lax.broadcasted_iota(jnp.int32, (SC_C, rows, RC), 0)
    pos4_ref[...] = NBH * chunk + pos[None]
    t = (jax.lax.broadcasted_iota(jnp.int32, (8, 64), 0) * 64
         + jax.lax.broadcasted_iota(jnp.int32, (8, 64), 1))
    te_ref[...] = ((t * TE) >= f_base).astype(jnp.int32)


def _expert_body(te_ref, x0_ref, x1_ref, x2_ref, x3_ref,
                 fw1_ref, fb1_ref, fg_ref, fbeta_ref, fw2_ref, fb2_ref,
                 rw1_ref, rb1_ref, rg_ref, rbeta_ref, rw2_ref, rb2_ref,
                 o_ref):
    e = te_ref[pl.program_id(0)]
    x_refs = (x0_ref, x1_ref, x2_ref, x3_ref)

    def head(w1_ref, b1_ref, g_ref, beta_ref, w2_ref, b2_ref):
        h = jnp.dot(x_refs[0][...], w1_ref[0:DC, :],
                    preferred_element_type=jnp.float32)
        for c in range(1, SC_C):
            h = h + jnp.dot(x_refs[c][...], w1_ref[c * DC:(c + 1) * DC, :],
                            preferred_element_type=jnp.float32)
        h = _ln_relu(h + b1_ref[...], g_ref[...], beta_ref[...])
        l = jnp.dot(h, w2_ref[...], preferred_element_type=jnp.float32)
        l = l + b2_ref[...]
        o_ref[...] = jnp.concatenate(
            [l, jnp.zeros((TE, OW - 2), jnp.float32)], axis=1)

    @pl.when(e == 0)
    def _():
        head(fw1_ref, fb1_ref, fg_ref, fbeta_ref, fw2_ref, fb2_ref)

    @pl.when(e != 0)
    def _():
        head(rw1_ref, rb1_ref, rg_ref, rbeta_ref, rw2_ref, rb2_ref)


def _sc_mesh():
    return plsc.VectorSubcoreMesh(core_axis_name="core",
                                  subcore_axis_name="subcore")


def _sc_scatter_rows(x, pos4, half):
    base = half * (H // SC_W)

    @pl.kernel(out_type=jax.ShapeDtypeStruct((SC_C * NBH, DC), jnp.float32),
               mesh=_sc_mesh())
    def k(x_hbm, i_hbm, o_hbm):
        def body(x_vmem, i_vmem):
            pltpu.sync_copy(x_vmem, o_hbm.at[i_vmem.at[0, 0]])

        pltpu.emit_pipeline(
            body,
            grid=(H // SC_W, SC_C),
            in_specs=[pl.BlockSpec((SC_W, DC), lambda i, c: (base + i, c)),
                      pl.BlockSpec((1, 1, SC_W), lambda i, c: (c, i, 0))],
            out_specs=[],
            core_axis_name=("core", "subcore"),
            dimension_semantics=(pltpu.PARALLEL, pltpu.PARALLEL),
        )(x_hbm, i_hbm)

    return k(x, pos4)


def _sc_gather_rows(data, pos):
    @pl.kernel(out_type=jax.ShapeDtypeStruct((H, OW), jnp.float32),
               mesh=_sc_mesh())
    def k(d_hbm, i_hbm, o_hbm):
        def body(i_vmem, o_vmem):
            pltpu.sync_copy(d_hbm.at[i_vmem.at[0]], o_vmem)

        pltpu.emit_pipeline(
            body,
            grid=(H // SC_GW,),
            in_specs=[pl.BlockSpec((1, SC_GW), lambda i: (i, 0))],
            out_specs=[pl.BlockSpec((SC_GW, OW), lambda i: (i, 0))],
            core_axis_name=("core", "subcore"),
            dimension_semantics=(pltpu.PARALLEL,),
        )(i_hbm, o_hbm)

    return k(data, pos)


def kernel(x,
           first_W1, first_b1, first_g, first_beta, first_W2, first_b2,
           fake_W1, fake_b1, fake_g, fake_beta, fake_W2, fake_b2,
           real_W1, real_b1, real_g, real_beta, real_W2, real_b2):
    f32 = jnp.float32
    full = lambda shape: pl.BlockSpec(shape, lambda i: (0,) * len(shape))

    def router(half):
        base = half * (H // TR)
        return pl.pallas_call(
            _router_body,
            grid=(H // TR,),
            in_specs=[pl.BlockSpec((TR, D), lambda i: (base + i, 0)),
                      full((D, D)), full((1, D)), full((1, D)), full((1, D)),
                      full((D, 2)), full((1, 2))],
            out_specs=pl.BlockSpec((TR // RC, RC), lambda i: (i, 0)),
            out_shape=jax.ShapeDtypeStruct((RWH, RC), jnp.int32),
            compiler_params=_PAR,
        )(x, first_W1, first_b1[None], first_g[None], first_beta[None],
          first_W2, first_b2[None])

    def positions(routes):
        return pl.pallas_call(
            _pos_body,
            out_shape=[jax.ShapeDtypeStruct((RWH, RC), jnp.int32),
                       jax.ShapeDtypeStruct((SC_C, RWH, RC), jnp.int32),
                       jax.ShapeDtypeStruct((8, 64), jnp.int32)],
        )(routes)

    def expert(xs, tile_expert):
        grid_spec = pltpu.PrefetchScalarGridSpec(
            num_scalar_prefetch=1,
            grid=(NTH,),
            in_specs=[
                pl.BlockSpec((TE, DC), lambda i, te: (0 * NTH + i, 0)),
                pl.BlockSpec((TE, DC), lambda i, te: (1 * NTH + i, 0)),
                pl.BlockSpec((TE, DC), lambda i, te: (2 * NTH + i, 0)),
                pl.BlockSpec((TE, DC), lambda i, te: (3 * NTH + i, 0)),
                pl.BlockSpec((D, D), lambda i, te: (0, 0)),
                pl.BlockSpec((1, D), lambda i, te: (0, 0)),
                pl.BlockSpec((1, D), lambda i, te: (0, 0)),
                pl.BlockSpec((1, D), lambda i, te: (0, 0)),
                pl.BlockSpec((D, 2), lambda i, te: (0, 0)),
                pl.BlockSpec((1, 2), lambda i, te: (0, 0)),
                pl.BlockSpec((D, D), lambda i, te: (0, 0)),
                pl.BlockSpec((1, D), lambda i, te: (0, 0)),
                pl.BlockSpec((1, D), lambda i, te: (0, 0)),
                pl.BlockSpec((1, D), lambda i, te: (0, 0)),
                pl.BlockSpec((D, 2), lambda i, te: (0, 0)),
                pl.BlockSpec((1, 2), lambda i, te: (0, 0)),
            ],
            out_specs=pl.BlockSpec((TE, OW), lambda i, te: (i, 0)),
        )
        return pl.pallas_call(
            _expert_body,
            grid_spec=grid_spec,
            out_shape=jax.ShapeDtypeStruct((NBH, OW), f32),
            compiler_params=pltpu.CompilerParams(
                dimension_semantics=("arbitrary",)),
        )(tile_expert, xs, xs, xs, xs,
          fake_W1, fake_b1[None], fake_g[None], fake_beta[None],
          fake_W2, fake_b2[None],
          real_W1, real_b1[None], real_g[None], real_beta[None],
          real_W2, real_b2[None])

    routes_h, pos_h, xs_h = [], [], []
    for h in range(NH):
        routes = router(h)
        pos2, pos4, te2 = positions(routes)
        routes_h.append(routes)
        pos_h.append((pos2, te2))
        xs_h.append(_sc_scatter_rows(x, pos4, h))

    fin_h = []
    for h in range(NH):
        pos2, te2 = pos_h[h]
        out_sorted = expert(xs_h[h], te2.reshape(8 * 64)[:NTH])
        fin_h.append(_sc_gather_rows(out_sorted, pos2)[:, :2])

    route_out = jnp.concatenate([r.reshape(H) for r in routes_h])
    return route_out, jnp.concatenate(fin_h, axis=0)
